# baseline (speedup 1.0000x reference)
import functools

import jax
import jax.numpy as jnp
from jax.experimental import pallas as pl
from jax.experimental.pallas import tpu as pltpu

T = 2048
D = 768
F = 3072
E = 8
FB = 768
NF = F // FB


def _routing_body(x_ref, ls_ref, lb_ref, wr_ref, br_ref, gates_ref):
    x = x_ref[...]
    mu = jnp.mean(x, axis=1, keepdims=True)
    xc = x - mu
    var = jnp.mean(xc * xc, axis=1, keepdims=True)
    h = xc / jnp.sqrt(var + 1e-5) * ls_ref[...] + lb_ref[...]
    logits = jnp.dot(h.astype(jnp.bfloat16), wr_ref[...].astype(jnp.bfloat16),
                     preferred_element_type=jnp.float32) + br_ref[...]
    iota_e = jax.lax.broadcasted_iota(jnp.int32, logits.shape, 1)
    v1 = jnp.max(logits, axis=1, keepdims=True)
    i1 = jnp.argmax(logits, axis=1)[:, None]
    masked = jnp.where(iota_e == i1, -jnp.inf, logits)
    v2 = jnp.max(masked, axis=1, keepdims=True)
    i2 = jnp.argmax(masked, axis=1)[:, None]
    t = jnp.exp(v2 - v1)
    g1 = 1.0 / (1.0 + t)
    g2 = t / (1.0 + t)
    gates_ref[...] = (jnp.where(iota_e == i1, g1, 0.0)
                      + jnp.where(iota_e == i2, g2, 0.0))


def _routing(x, ln_scale, ln_bias, Wr, br):
    return pl.pallas_call(
        _routing_body,
        grid=(8,),
        in_specs=[
            pl.BlockSpec((T // 8, D), lambda i: (i, 0)),
            pl.BlockSpec((1, D), lambda i: (0, 0)),
            pl.BlockSpec((1, D), lambda i: (0, 0)),
            pl.BlockSpec((D, E), lambda i: (0, 0)),
            pl.BlockSpec((1, E), lambda i: (0, 0)),
        ],
        out_specs=pl.BlockSpec((T // 8, E), lambda i: (i, 0)),
        out_shape=jax.ShapeDtypeStruct((T, E), jnp.float32),
    )(x, ln_scale.reshape(1, D), ln_bias.reshape(1, D), Wr, br.reshape(1, E))


def _ffn_body(x_ref, g_ref, w1_ref, b1_ref, w2_ref, b2_ref, y_ref):
    e = pl.program_id(0)
    f = pl.program_id(1)

    @pl.when((e == 0) & (f == 0))
    def _():
        y_ref[...] = jnp.zeros_like(y_ref)

    xb = x_ref[...].astype(jnp.bfloat16)
    h = jnp.dot(xb, w1_ref[0], preferred_element_type=jnp.float32) + b1_ref[0]
    h = jax.nn.gelu(h)
    o = jnp.dot(h.astype(jnp.bfloat16), w2_ref[0],
                preferred_element_type=jnp.float32)
    o = jnp.where(f == NF - 1, 1.0, 0.0) * b2_ref[0] + o
    gfull = g_ref[...]
    iota_e = jax.lax.broadcasted_iota(jnp.int32, gfull.shape, 1)
    gcol = jnp.sum(jnp.where(iota_e == e, gfull, 0.0), axis=1, keepdims=True)
    y_ref[...] += o * gcol


def _ffn(x, gates, W1, b1, W2, b2):
    return pl.pallas_call(
        _ffn_body,
        grid=(E, NF),
        in_specs=[
            pl.BlockSpec((T, D), lambda e, f: (0, 0)),
            pl.BlockSpec((T, E), lambda e, f: (0, 0)),
            pl.BlockSpec((1, D, FB), lambda e, f: (e, 0, f)),
            pl.BlockSpec((1, 1, FB), lambda e, f: (e, 0, f)),
            pl.BlockSpec((1, FB, D), lambda e, f: (e, f, 0)),
            pl.BlockSpec((1, 1, D), lambda e, f: (e, 0, 0)),
        ],
        out_specs=pl.BlockSpec((T, D), lambda e, f: (0, 0)),
        out_shape=jax.ShapeDtypeStruct((T, D), jnp.float32),
    )(x, gates, W1, b1.reshape(E, 1, F), W2, b2.reshape(E, 1, D))


def kernel(x, ln_scale, ln_bias, Wr, br, W1, b1, W2, b2):
    gates = _routing(x, ln_scale, ln_bias, Wr, br)
    W1b = W1.astype(jnp.bfloat16)
    W2b = W2.astype(jnp.bfloat16)
    return _ffn(x, gates, W1b, b1, W2b, b2)


# speedup vs baseline: 1.2617x; 1.2617x over previous
import functools

import jax
import jax.numpy as jnp
from jax import lax
from jax.experimental import pallas as pl
from jax.experimental.pallas import tpu as pltpu
from jax.experimental.pallas import tpu_sc as plsc

T = 2048
D = 768
F = 3072
E = 8
RB = 128
NT = 40
SP = NT * RB
NTMAP = 48
NSUB = 16
NCORE = 2

_mesh = functools.partial(plsc.VectorSubcoreMesh, core_axis_name="c",
                          subcore_axis_name="s", num_cores=NCORE,
                          num_subcores=NSUB)


def _routing_body(x_ref, ls_ref, lb_ref, wr_ref, br_ref,
                  i1_ref, i2_ref, g1_ref, g2_ref):
    x = x_ref[...]
    mu = jnp.mean(x, axis=1, keepdims=True)
    xc = x - mu
    var = jnp.mean(xc * xc, axis=1, keepdims=True)
    h = xc / jnp.sqrt(var + 1e-5) * ls_ref[...] + lb_ref[...]
    logits = jnp.dot(h.astype(jnp.bfloat16), wr_ref[...].astype(jnp.bfloat16),
                     preferred_element_type=jnp.float32) + br_ref[...]
    iota_e = jax.lax.broadcasted_iota(jnp.int32, logits.shape, 1)
    v1 = jnp.max(logits, axis=1, keepdims=True)
    i1 = jnp.argmax(logits, axis=1)[:, None]
    masked = jnp.where(iota_e == i1, -jnp.inf, logits)
    v2 = jnp.max(masked, axis=1, keepdims=True)
    i2 = jnp.argmax(masked, axis=1)[:, None]
    t = jnp.exp(v2 - v1)
    i1_ref[...] = i1
    i2_ref[...] = i2
    g1_ref[...] = 1.0 / (1.0 + t)
    g2_ref[...] = t / (1.0 + t)


def _routing(x, ln_scale, ln_bias, Wr, br):
    tb = T // 8
    return pl.pallas_call(
        _routing_body,
        grid=(8,),
        in_specs=[
            pl.BlockSpec((tb, D), lambda i: (i, 0)),
            pl.BlockSpec((1, D), lambda i: (0, 0)),
            pl.BlockSpec((1, D), lambda i: (0, 0)),
            pl.BlockSpec((D, E), lambda i: (0, 0)),
            pl.BlockSpec((1, E), lambda i: (0, 0)),
        ],
        out_specs=[
            pl.BlockSpec((tb, 1), lambda i: (i, 0)),
            pl.BlockSpec((tb, 1), lambda i: (i, 0)),
            pl.BlockSpec((tb, 1), lambda i: (i, 0)),
            pl.BlockSpec((tb, 1), lambda i: (i, 0)),
        ],
        out_shape=[
            jax.ShapeDtypeStruct((T, 1), jnp.int32),
            jax.ShapeDtypeStruct((T, 1), jnp.int32),
            jax.ShapeDtypeStruct((T, 1), jnp.float32),
            jax.ShapeDtypeStruct((T, 1), jnp.float32),
        ],
    )(x, ln_scale.reshape(1, D), ln_bias.reshape(1, D), Wr, br.reshape(1, E))


def _dispatch_body(i1h, i2h, p1h, p2h, teh, tah,
                   ex_v, cnt_v, call_v, pos_v, tile_v, act_v, sh_cnt):
    c = lax.axis_index("c")
    s = lax.axis_index("s")

    @pl.when(c == 0)
    def _():
        k_is1 = s >= 8
        t0 = jnp.where(k_is1, (s - 8) * 256, s * 256)

        @pl.when(k_is1)
        def _():
            pltpu.sync_copy(i2h.at[pl.ds(t0, 256)], ex_v)

        @pl.when(jnp.logical_not(k_is1))
        def _():
            pltpu.sync_copy(i1h.at[pl.ds(t0, 256)], ex_v)

        lane = lax.iota(jnp.int32, 16)
        zeros = jnp.zeros((16,), jnp.int32)
        counts = zeros
        vregs = []
        for i in range(16):
            v = ex_v[pl.ds(i * 16, 16)]
            vregs.append(v)
            for e in range(E):
                m = v == e
                cpop = jnp.sum(jnp.where(m, 1, 0))
                counts = counts + jnp.where(lane == e, cpop, 0)
        cnt_v[...] = counts
        pltpu.sync_copy(cnt_v, sh_cnt.at[pl.ds(s * 16, 16)])
        plsc.subcore_barrier()
        pltpu.sync_copy(sh_cnt, call_v)

        total = zeros
        prefix = zeros
        for w in range(NSUB):
            row = call_v[pl.ds(w * 16, 16)]
            total = total + row
            wmask = jnp.full((16,), w, jnp.int32) < s
            prefix = prefix + jnp.where(wmask, row, 0)
        ntiles = (total + (RB - 1)) // RB
        cum = plsc.cumsum(ntiles)
        base = (cum - ntiles) * RB
        running = base + prefix

        for i in range(16):
            v = vregs[i]
            posv = zeros
            for e in range(E):
                m = v == e
                rank = plsc.cumsum(jnp.where(m, 1, 0))
                b_e = jnp.sum(jnp.where(lane == e, running, 0))
                posv = jnp.where(m, b_e + rank - 1, posv)
                cpop = jnp.sum(jnp.where(m, 1, 0))
                running = running + jnp.where(lane == e, cpop, 0)
            pos_v[pl.ds(i * 16, 16)] = posv

        @pl.when(k_is1)
        def _():
            pltpu.sync_copy(pos_v, p2h.at[pl.ds(t0, 256)])

        @pl.when(jnp.logical_not(k_is1))
        def _():
            pltpu.sync_copy(pos_v, p1h.at[pl.ds(t0, 256)])

        @pl.when(s == 0)
        def _():
            tot_tiles = jnp.sum(jnp.where(lane == 7, cum, 0))
            cs = [jnp.sum(jnp.where(lane == e, cum, 0)) for e in range(E)]
            el = jnp.int32(0)
            for e in range(E):
                el = el + jnp.where(tot_tiles - 1 >= cs[e], 1, 0)
            el = jnp.minimum(el, E - 1)
            for ch in range(NTMAP // 16):
                jv = lane + ch * 16
                acc = zeros
                for e in range(E):
                    acc = acc + jnp.where(jv >= cs[e], 1, 0)
                active = jnp.where(jv < tot_tiles, 1, 0)
                expert = jnp.where(jv < tot_tiles, jnp.minimum(acc, E - 1), el)
                tile_v[pl.ds(ch * 16, 16)] = expert
                act_v[pl.ds(ch * 16, 16)] = active
            pltpu.sync_copy(tile_v, teh)
            pltpu.sync_copy(act_v, tah)


def _dispatch(i1, i2):
    return pl.kernel(
        _dispatch_body,
        out_type=[
            jax.ShapeDtypeStruct((T,), jnp.int32),
            jax.ShapeDtypeStruct((T,), jnp.int32),
            jax.ShapeDtypeStruct((NTMAP,), jnp.int32),
            jax.ShapeDtypeStruct((NTMAP,), jnp.int32),
        ],
        mesh=_mesh(),
        compiler_params=pltpu.CompilerParams(needs_layout_passes=False),
        scratch_types=[
            pltpu.VMEM((256,), jnp.int32),
            pltpu.VMEM((16,), jnp.int32),
            pltpu.VMEM((NSUB * 16,), jnp.int32),
            pltpu.VMEM((256,), jnp.int32),
            pltpu.VMEM((NTMAP,), jnp.int32),
            pltpu.VMEM((NTMAP,), jnp.int32),
            pltpu.VMEM_SHARED((NSUB * 16,), jnp.int32),
        ],
    )(i1, i2)


def _xscatter_body(xh, p1h, p2h, xsh, idx_v, rows_v, sem):
    c = lax.axis_index("c")
    s = lax.axis_index("s")
    wid = s * NCORE + c
    k_is1 = wid >= 16
    t0 = jnp.where(k_is1, (wid - 16) * 128, wid * 128)

    @pl.when(k_is1)
    def _():
        pltpu.sync_copy(p2h.at[pl.ds(t0, 128)], idx_v)

    @pl.when(jnp.logical_not(k_is1))
    def _():
        pltpu.sync_copy(p1h.at[pl.ds(t0, 128)], idx_v)

    pltpu.sync_copy(xh.at[pl.ds(t0, 128)], rows_v)
    pltpu.async_copy(rows_v, xsh.at[idx_v], sem).wait()


def _xscatter(x, pos1, pos2):
    return pl.kernel(
        _xscatter_body,
        out_type=jax.ShapeDtypeStruct((SP, D), jnp.float32),
        mesh=_mesh(),
        compiler_params=pltpu.CompilerParams(needs_layout_passes=False),
        scratch_types=[
            pltpu.VMEM((128,), jnp.int32),
            pltpu.VMEM((128, D), jnp.float32),
            pltpu.SemaphoreType.DMA,
        ],
    )(x, pos1, pos2)


def _ffn_body(te_ref, ta_ref, x_ref, w1_ref, b1_ref, w2_ref, b2_ref, out_ref):
    j = pl.program_id(0)

    @pl.when(ta_ref[j] == 1)
    def _():
        xb = x_ref[...].astype(jnp.bfloat16)
        h = jnp.dot(xb, w1_ref[0], preferred_element_type=jnp.float32)
        h = jax.nn.gelu(h + b1_ref[0])
        o = jnp.dot(h.astype(jnp.bfloat16), w2_ref[0],
                    preferred_element_type=jnp.float32)
        out_ref[...] = o + b2_ref[0]


def _ffn_grouped(xs, te, ta, W1b, b1, W2b, b2):
    grid_spec = pltpu.PrefetchScalarGridSpec(
        num_scalar_prefetch=2,
        grid=(NT,),
        in_specs=[
            pl.BlockSpec((RB, D), lambda j, te, ta: (j, 0)),
            pl.BlockSpec((1, D, F), lambda j, te, ta: (te[j], 0, 0)),
            pl.BlockSpec((1, 1, F), lambda j, te, ta: (te[j], 0, 0)),
            pl.BlockSpec((1, F, D), lambda j, te, ta: (te[j], 0, 0)),
            pl.BlockSpec((1, 1, D), lambda j, te, ta: (te[j], 0, 0)),
        ],
        out_specs=pl.BlockSpec((RB, D), lambda j, te, ta: (j, 0)),
    )
    return pl.pallas_call(
        _ffn_body,
        grid_spec=grid_spec,
        out_shape=jax.ShapeDtypeStruct((SP, D), jnp.float32),
    )(te, ta, xs, W1b, b1.reshape(E, 1, F), W2b, b2.reshape(E, 1, D))


def _combine_body(oh, p1h, p2h, g1h, g2h, yh,
                  pa, pb, ga, gb, A, B, sem_a, sem_b):
    c = lax.axis_index("c")
    s = lax.axis_index("s")
    wid = s * NCORE + c
    t0 = wid * (T // 32)
    n = T // 32
    pltpu.sync_copy(p1h.at[pl.ds(t0, n)], pa)
    pltpu.sync_copy(p2h.at[pl.ds(t0, n)], pb)
    pltpu.sync_copy(g1h.at[pl.ds(t0, n)], ga)
    pltpu.sync_copy(g2h.at[pl.ds(t0, n)], gb)
    cpa = pltpu.async_copy(oh.at[pa], A, sem_a)
    cpb = pltpu.async_copy(oh.at[pb], B, sem_b)
    cpa.wait()
    cpb.wait()

    def body(g, carry):
        ga16 = ga[pl.ds(g * 16, 16)]
        gb16 = gb[pl.ds(g * 16, 16)]
        for tk in range(16):
            i = g * 16 + tk
            gav = jnp.full((16,), ga16[tk])
            gbv = jnp.full((16,), gb16[tk])
            for jj in range(D // 16):
                sl = pl.ds(jj * 16, 16)
                A[i, sl] = gav * A[i, sl] + gbv * B[i, sl]
        return carry

    lax.fori_loop(0, n // 16, body, 0)
    pltpu.sync_copy(A, yh.at[pl.ds(t0, n)])


def _combine(outs, pos1, pos2, g1, g2):
    n = T // 32
    return pl.kernel(
        _combine_body,
        out_type=jax.ShapeDtypeStruct((T, D), jnp.float32),
        mesh=_mesh(),
        compiler_params=pltpu.CompilerParams(needs_layout_passes=False),
        scratch_types=[
            pltpu.VMEM((n,), jnp.int32),
            pltpu.VMEM((n,), jnp.int32),
            pltpu.VMEM((n,), jnp.float32),
            pltpu.VMEM((n,), jnp.float32),
            pltpu.VMEM((n, D), jnp.float32),
            pltpu.VMEM((n, D), jnp.float32),
            pltpu.SemaphoreType.DMA,
            pltpu.SemaphoreType.DMA,
        ],
    )(outs, pos1, pos2, g1, g2)


def kernel(x, ln_scale, ln_bias, Wr, br, W1, b1, W2, b2):
    i1, i2, g1, g2 = _routing(x, ln_scale, ln_bias, Wr, br)
    i1 = i1.reshape(T)
    i2 = i2.reshape(T)
    pos1, pos2, te, ta = _dispatch(i1, i2)
    xs = _xscatter(x, pos1, pos2)
    W1b = W1.astype(jnp.bfloat16)
    W2b = W2.astype(jnp.bfloat16)
    outs = _ffn_grouped(xs, te, ta, W1b, b1, W2b, b2)
    return _combine(outs, pos1, pos2, g1.reshape(T), g2.reshape(T))


# speedup vs baseline: 1.3196x; 1.0459x over previous
import functools

import jax
import jax.numpy as jnp
from jax import lax
from jax.experimental import pallas as pl
from jax.experimental.pallas import tpu as pltpu
from jax.experimental.pallas import tpu_sc as plsc

T = 2048
D = 768
F = 3072
E = 8
RB = 256
NT = 24
SP = NT * RB
NTMAP = 48
NSUB = 16
NCORE = 2

_mesh = functools.partial(plsc.VectorSubcoreMesh, core_axis_name="c",
                          subcore_axis_name="s", num_cores=NCORE,
                          num_subcores=NSUB)


def _routing_body(x_ref, ls_ref, lb_ref, wr_ref, br_ref,
                  i1_ref, i2_ref, g1_ref, g2_ref):
    x = x_ref[...]
    mu = jnp.mean(x, axis=1, keepdims=True)
    xc = x - mu
    var = jnp.mean(xc * xc, axis=1, keepdims=True)
    h = xc / jnp.sqrt(var + 1e-5) * ls_ref[...] + lb_ref[...]
    logits = jnp.dot(h.astype(jnp.bfloat16), wr_ref[...].astype(jnp.bfloat16),
                     preferred_element_type=jnp.float32) + br_ref[...]
    iota_e = jax.lax.broadcasted_iota(jnp.int32, logits.shape, 1)
    v1 = jnp.max(logits, axis=1, keepdims=True)
    i1 = jnp.argmax(logits, axis=1)[:, None]
    masked = jnp.where(iota_e == i1, -jnp.inf, logits)
    v2 = jnp.max(masked, axis=1, keepdims=True)
    i2 = jnp.argmax(masked, axis=1)[:, None]
    t = jnp.exp(v2 - v1)
    i1_ref[...] = i1
    i2_ref[...] = i2
    g1_ref[...] = 1.0 / (1.0 + t)
    g2_ref[...] = t / (1.0 + t)


def _routing(x, ln_scale, ln_bias, Wr, br):
    tb = T // 8
    return pl.pallas_call(
        _routing_body,
        grid=(8,),
        in_specs=[
            pl.BlockSpec((tb, D), lambda i: (i, 0)),
            pl.BlockSpec((1, D), lambda i: (0, 0)),
            pl.BlockSpec((1, D), lambda i: (0, 0)),
            pl.BlockSpec((D, E), lambda i: (0, 0)),
            pl.BlockSpec((1, E), lambda i: (0, 0)),
        ],
        out_specs=[
            pl.BlockSpec((tb, 1), lambda i: (i, 0)),
            pl.BlockSpec((tb, 1), lambda i: (i, 0)),
            pl.BlockSpec((tb, 1), lambda i: (i, 0)),
            pl.BlockSpec((tb, 1), lambda i: (i, 0)),
        ],
        out_shape=[
            jax.ShapeDtypeStruct((T, 1), jnp.int32),
            jax.ShapeDtypeStruct((T, 1), jnp.int32),
            jax.ShapeDtypeStruct((T, 1), jnp.float32),
            jax.ShapeDtypeStruct((T, 1), jnp.float32),
        ],
    )(x, ln_scale.reshape(1, D), ln_bias.reshape(1, D), Wr, br.reshape(1, E))


def _dispatch_body(i1h, i2h, p1h, p2h, teh, tah,
                   ex_v, cnt_v, call_v, pos_v, tile_v, act_v, sh_cnt):
    c = lax.axis_index("c")
    s = lax.axis_index("s")

    @pl.when(c == 0)
    def _():
        k_is1 = s >= 8
        t0 = jnp.where(k_is1, (s - 8) * 256, s * 256)

        @pl.when(k_is1)
        def _():
            pltpu.sync_copy(i2h.at[pl.ds(t0, 256)], ex_v)

        @pl.when(jnp.logical_not(k_is1))
        def _():
            pltpu.sync_copy(i1h.at[pl.ds(t0, 256)], ex_v)

        lane = lax.iota(jnp.int32, 16)
        zeros = jnp.zeros((16,), jnp.int32)
        counts = zeros
        vregs = []
        for i in range(16):
            v = ex_v[pl.ds(i * 16, 16)]
            vregs.append(v)
            for e in range(E):
                m = v == e
                cpop = jnp.sum(jnp.where(m, 1, 0))
                counts = counts + jnp.where(lane == e, cpop, 0)
        cnt_v[...] = counts
        pltpu.sync_copy(cnt_v, sh_cnt.at[pl.ds(s * 16, 16)])
        plsc.subcore_barrier()
        pltpu.sync_copy(sh_cnt, call_v)

        total = zeros
        prefix = zeros
        for w in range(NSUB):
            row = call_v[pl.ds(w * 16, 16)]
            total = total + row
            wmask = jnp.full((16,), w, jnp.int32) < s
            prefix = prefix + jnp.where(wmask, row, 0)
        ntiles = (total + (RB - 1)) // RB
        cum = plsc.cumsum(ntiles)
        base = (cum - ntiles) * RB
        running = base + prefix

        for i in range(16):
            v = vregs[i]
            posv = zeros
            for e in range(E):
                m = v == e
                rank = plsc.cumsum(jnp.where(m, 1, 0))
                b_e = jnp.sum(jnp.where(lane == e, running, 0))
                posv = jnp.where(m, b_e + rank - 1, posv)
                cpop = jnp.sum(jnp.where(m, 1, 0))
                running = running + jnp.where(lane == e, cpop, 0)
            pos_v[pl.ds(i * 16, 16)] = posv

        @pl.when(k_is1)
        def _():
            pltpu.sync_copy(pos_v, p2h.at[pl.ds(t0, 256)])

        @pl.when(jnp.logical_not(k_is1))
        def _():
            pltpu.sync_copy(pos_v, p1h.at[pl.ds(t0, 256)])

        @pl.when(s == 0)
        def _():
            tot_tiles = jnp.sum(jnp.where(lane == 7, cum, 0))
            cs = [jnp.sum(jnp.where(lane == e, cum, 0)) for e in range(E)]
            el = jnp.int32(0)
            for e in range(E):
                el = el + jnp.where(tot_tiles - 1 >= cs[e], 1, 0)
            el = jnp.minimum(el, E - 1)
            for ch in range(NTMAP // 16):
                jv = lane + ch * 16
                acc = zeros
                for e in range(E):
                    acc = acc + jnp.where(jv >= cs[e], 1, 0)
                active = jnp.where(jv < tot_tiles, 1, 0)
                expert = jnp.where(jv < tot_tiles, jnp.minimum(acc, E - 1), el)
                tile_v[pl.ds(ch * 16, 16)] = expert
                act_v[pl.ds(ch * 16, 16)] = active
            pltpu.sync_copy(tile_v, teh)
            pltpu.sync_copy(act_v, tah)


def _dispatch(i1, i2):
    return pl.kernel(
        _dispatch_body,
        out_type=[
            jax.ShapeDtypeStruct((T,), jnp.int32),
            jax.ShapeDtypeStruct((T,), jnp.int32),
            jax.ShapeDtypeStruct((NTMAP,), jnp.int32),
            jax.ShapeDtypeStruct((NTMAP,), jnp.int32),
        ],
        mesh=_mesh(),
        compiler_params=pltpu.CompilerParams(needs_layout_passes=False),
        scratch_types=[
            pltpu.VMEM((256,), jnp.int32),
            pltpu.VMEM((16,), jnp.int32),
            pltpu.VMEM((NSUB * 16,), jnp.int32),
            pltpu.VMEM((256,), jnp.int32),
            pltpu.VMEM((NTMAP,), jnp.int32),
            pltpu.VMEM((NTMAP,), jnp.int32),
            pltpu.VMEM_SHARED((NSUB * 16,), jnp.int32),
        ],
    )(i1, i2)


def _xscatter_body(xh, p1h, p2h, xsh, idx_v, rows_v, sem):
    c = lax.axis_index("c")
    s = lax.axis_index("s")
    wid = s * NCORE + c
    k_is1 = wid >= 16
    t0 = jnp.where(k_is1, (wid - 16) * 128, wid * 128)

    @pl.when(k_is1)
    def _():
        pltpu.sync_copy(p2h.at[pl.ds(t0, 128)], idx_v)

    @pl.when(jnp.logical_not(k_is1))
    def _():
        pltpu.sync_copy(p1h.at[pl.ds(t0, 128)], idx_v)

    pltpu.sync_copy(xh.at[pl.ds(t0, 128)], rows_v)
    pltpu.async_copy(rows_v, xsh.at[idx_v], sem).wait()


def _xscatter(x, pos1, pos2):
    return pl.kernel(
        _xscatter_body,
        out_type=jax.ShapeDtypeStruct((SP, D), jnp.float32),
        mesh=_mesh(),
        compiler_params=pltpu.CompilerParams(needs_layout_passes=False),
        scratch_types=[
            pltpu.VMEM((128,), jnp.int32),
            pltpu.VMEM((128, D), jnp.float32),
            pltpu.SemaphoreType.DMA,
        ],
    )(x, pos1, pos2)


def _ffn_body(te_ref, ta_ref, x_ref, w1_ref, b1_ref, w2_ref, b2_ref, out_ref):
    j = pl.program_id(0)

    @pl.when(ta_ref[j] == 1)
    def _():
        xb = x_ref[...].astype(jnp.bfloat16)
        h = jnp.dot(xb, w1_ref[0], preferred_element_type=jnp.float32)
        h = jax.nn.gelu(h + b1_ref[0])
        o = jnp.dot(h.astype(jnp.bfloat16), w2_ref[0],
                    preferred_element_type=jnp.float32)
        out_ref[...] = o + b2_ref[0]


def _ffn_grouped(xs, te, ta, W1b, b1, W2b, b2):
    grid_spec = pltpu.PrefetchScalarGridSpec(
        num_scalar_prefetch=2,
        grid=(NT,),
        in_specs=[
            pl.BlockSpec((RB, D), lambda j, te, ta: (j, 0)),
            pl.BlockSpec((1, D, F), lambda j, te, ta: (te[j], 0, 0)),
            pl.BlockSpec((1, 1, F), lambda j, te, ta: (te[j], 0, 0)),
            pl.BlockSpec((1, F, D), lambda j, te, ta: (te[j], 0, 0)),
            pl.BlockSpec((1, 1, D), lambda j, te, ta: (te[j], 0, 0)),
        ],
        out_specs=pl.BlockSpec((RB, D), lambda j, te, ta: (j, 0)),
    )
    return pl.pallas_call(
        _ffn_body,
        grid_spec=grid_spec,
        out_shape=jax.ShapeDtypeStruct((SP, D), jnp.float32),
    )(te, ta, xs, W1b, b1.reshape(E, 1, F), W2b, b2.reshape(E, 1, D))


def _combine_body(oh, p1h, p2h, g1h, g2h, yh,
                  pa, pb, ga, gb, A, B, sem_a, sem_b):
    c = lax.axis_index("c")
    s = lax.axis_index("s")
    wid = s * NCORE + c
    t0 = wid * (T // 32)
    n = T // 32
    pltpu.sync_copy(p1h.at[pl.ds(t0, n)], pa)
    pltpu.sync_copy(p2h.at[pl.ds(t0, n)], pb)
    pltpu.sync_copy(g1h.at[pl.ds(t0, n)], ga)
    pltpu.sync_copy(g2h.at[pl.ds(t0, n)], gb)
    cpa = pltpu.async_copy(oh.at[pa], A, sem_a)
    cpb = pltpu.async_copy(oh.at[pb], B, sem_b)
    cpa.wait()
    cpb.wait()

    def body(g, carry):
        ga16 = ga[pl.ds(g * 16, 16)]
        gb16 = gb[pl.ds(g * 16, 16)]
        for tk in range(16):
            i = g * 16 + tk
            gav = jnp.full((16,), ga16[tk])
            gbv = jnp.full((16,), gb16[tk])
            for jj in range(D // 16):
                sl = pl.ds(jj * 16, 16)
                A[i, sl] = gav * A[i, sl] + gbv * B[i, sl]
        return carry

    lax.fori_loop(0, n // 16, body, 0)
    pltpu.sync_copy(A, yh.at[pl.ds(t0, n)])


def _combine(outs, pos1, pos2, g1, g2):
    n = T // 32
    return pl.kernel(
        _combine_body,
        out_type=jax.ShapeDtypeStruct((T, D), jnp.float32),
        mesh=_mesh(),
        compiler_params=pltpu.CompilerParams(needs_layout_passes=False),
        scratch_types=[
            pltpu.VMEM((n,), jnp.int32),
            pltpu.VMEM((n,), jnp.int32),
            pltpu.VMEM((n,), jnp.float32),
            pltpu.VMEM((n,), jnp.float32),
            pltpu.VMEM((n, D), jnp.float32),
            pltpu.VMEM((n, D), jnp.float32),
            pltpu.SemaphoreType.DMA,
            pltpu.SemaphoreType.DMA,
        ],
    )(outs, pos1, pos2, g1, g2)


def kernel(x, ln_scale, ln_bias, Wr, br, W1, b1, W2, b2):
    i1, i2, g1, g2 = _routing(x, ln_scale, ln_bias, Wr, br)
    i1 = i1.reshape(T)
    i2 = i2.reshape(T)
    pos1, pos2, te, ta = _dispatch(i1, i2)
    xs = _xscatter(x, pos1, pos2)
    W1b = W1.astype(jnp.bfloat16)
    W2b = W2.astype(jnp.bfloat16)
    outs = _ffn_grouped(xs, te, ta, W1b, b1, W2b, b2)
    return _combine(outs, pos1, pos2, g1.reshape(T), g2.reshape(T))


# speedup vs baseline: 1.6344x; 1.2386x over previous
import functools

import jax
import jax.numpy as jnp
from jax import lax
from jax.experimental import pallas as pl
from jax.experimental.pallas import tpu as pltpu
from jax.experimental.pallas import tpu_sc as plsc

T = 2048
D = 768
F = 3072
E = 8
RB = 256
NT = 24
SP = NT * RB
NTMAP = 48
NSUB = 16
NCORE = 2

_mesh = functools.partial(plsc.VectorSubcoreMesh, core_axis_name="c",
                          subcore_axis_name="s", num_cores=NCORE,
                          num_subcores=NSUB)


def _routing_body(x_ref, ls_ref, lb_ref, wr_ref, br_ref,
                  i1_ref, i2_ref, g1_ref, g2_ref):
    x = x_ref[...]
    mu = jnp.mean(x, axis=1, keepdims=True)
    xc = x - mu
    var = jnp.mean(xc * xc, axis=1, keepdims=True)
    h = xc / jnp.sqrt(var + 1e-5) * ls_ref[...] + lb_ref[...]
    logits = jnp.dot(h.astype(jnp.bfloat16), wr_ref[...].astype(jnp.bfloat16),
                     preferred_element_type=jnp.float32) + br_ref[...]
    iota_e = jax.lax.broadcasted_iota(jnp.int32, logits.shape, 1)
    v1 = jnp.max(logits, axis=1, keepdims=True)
    i1 = jnp.argmax(logits, axis=1)[:, None]
    masked = jnp.where(iota_e == i1, -jnp.inf, logits)
    v2 = jnp.max(masked, axis=1, keepdims=True)
    i2 = jnp.argmax(masked, axis=1)[:, None]
    t = jnp.exp(v2 - v1)
    i1_ref[...] = i1
    i2_ref[...] = i2
    g1_ref[...] = 1.0 / (1.0 + t)
    g2_ref[...] = t / (1.0 + t)


def _routing(x, ln_scale, ln_bias, Wr, br):
    tb = T // 8
    return pl.pallas_call(
        _routing_body,
        grid=(8,),
        in_specs=[
            pl.BlockSpec((tb, D), lambda i: (i, 0)),
            pl.BlockSpec((1, D), lambda i: (0, 0)),
            pl.BlockSpec((1, D), lambda i: (0, 0)),
            pl.BlockSpec((D, E), lambda i: (0, 0)),
            pl.BlockSpec((1, E), lambda i: (0, 0)),
        ],
        out_specs=[
            pl.BlockSpec((tb, 1), lambda i: (i, 0)),
            pl.BlockSpec((tb, 1), lambda i: (i, 0)),
            pl.BlockSpec((tb, 1), lambda i: (i, 0)),
            pl.BlockSpec((tb, 1), lambda i: (i, 0)),
        ],
        out_shape=[
            jax.ShapeDtypeStruct((T, 1), jnp.int32),
            jax.ShapeDtypeStruct((T, 1), jnp.int32),
            jax.ShapeDtypeStruct((T, 1), jnp.float32),
            jax.ShapeDtypeStruct((T, 1), jnp.float32),
        ],
    )(x, ln_scale.reshape(1, D), ln_bias.reshape(1, D), Wr, br.reshape(1, E))


def _dispatch_body(i1h, i2h, p1h, p2h, teh, tah,
                   ex_v, cnt_v, call_v, pos_v, tile_v, act_v, sh_cnt):
    c = lax.axis_index("c")
    s = lax.axis_index("s")

    @pl.when(c == 0)
    def _():
        k_is1 = s >= 8
        t0 = jnp.where(k_is1, (s - 8) * 256, s * 256)

        @pl.when(k_is1)
        def _():
            pltpu.sync_copy(i2h.at[pl.ds(t0, 256)], ex_v)

        @pl.when(jnp.logical_not(k_is1))
        def _():
            pltpu.sync_copy(i1h.at[pl.ds(t0, 256)], ex_v)

        lane = lax.iota(jnp.int32, 16)
        zeros = jnp.zeros((16,), jnp.int32)
        counts = zeros
        vregs = []
        for i in range(16):
            v = ex_v[pl.ds(i * 16, 16)]
            vregs.append(v)
            for e in range(E):
                m = v == e
                cpop = jnp.sum(jnp.where(m, 1, 0))
                counts = counts + jnp.where(lane == e, cpop, 0)
        cnt_v[...] = counts
        pltpu.sync_copy(cnt_v, sh_cnt.at[pl.ds(s * 16, 16)])
        plsc.subcore_barrier()
        pltpu.sync_copy(sh_cnt, call_v)

        total = zeros
        prefix = zeros
        for w in range(NSUB):
            row = call_v[pl.ds(w * 16, 16)]
            total = total + row
            wmask = jnp.full((16,), w, jnp.int32) < s
            prefix = prefix + jnp.where(wmask, row, 0)
        ntiles = (total + (RB - 1)) // RB
        cum = plsc.cumsum(ntiles)
        base = (cum - ntiles) * RB
        running = base + prefix

        for i in range(16):
            v = vregs[i]
            posv = zeros
            for e in range(E):
                m = v == e
                rank = plsc.cumsum(jnp.where(m, 1, 0))
                b_e = jnp.sum(jnp.where(lane == e, running, 0))
                posv = jnp.where(m, b_e + rank - 1, posv)
                cpop = jnp.sum(jnp.where(m, 1, 0))
                running = running + jnp.where(lane == e, cpop, 0)
            pos_v[pl.ds(i * 16, 16)] = posv

        @pl.when(k_is1)
        def _():
            pltpu.sync_copy(pos_v, p2h.at[pl.ds(t0, 256)])

        @pl.when(jnp.logical_not(k_is1))
        def _():
            pltpu.sync_copy(pos_v, p1h.at[pl.ds(t0, 256)])

        @pl.when(s == 0)
        def _():
            tot_tiles = jnp.sum(jnp.where(lane == 7, cum, 0))
            cs = [jnp.sum(jnp.where(lane == e, cum, 0)) for e in range(E)]
            el = jnp.int32(0)
            for e in range(E):
                el = el + jnp.where(tot_tiles - 1 >= cs[e], 1, 0)
            el = jnp.minimum(el, E - 1)
            for ch in range(NTMAP // 16):
                jv = lane + ch * 16
                acc = zeros
                for e in range(E):
                    acc = acc + jnp.where(jv >= cs[e], 1, 0)
                active = jnp.where(jv < tot_tiles, 1, 0)
                expert = jnp.where(jv < tot_tiles, jnp.minimum(acc, E - 1), el)
                tile_v[pl.ds(ch * 16, 16)] = expert
                act_v[pl.ds(ch * 16, 16)] = active
            pltpu.sync_copy(tile_v, teh)
            pltpu.sync_copy(act_v, tah)


def _dispatch(i1, i2):
    return pl.kernel(
        _dispatch_body,
        out_type=[
            jax.ShapeDtypeStruct((T,), jnp.int32),
            jax.ShapeDtypeStruct((T,), jnp.int32),
            jax.ShapeDtypeStruct((NTMAP,), jnp.int32),
            jax.ShapeDtypeStruct((NTMAP,), jnp.int32),
        ],
        mesh=_mesh(),
        compiler_params=pltpu.CompilerParams(needs_layout_passes=False),
        scratch_types=[
            pltpu.VMEM((256,), jnp.int32),
            pltpu.VMEM((16,), jnp.int32),
            pltpu.VMEM((NSUB * 16,), jnp.int32),
            pltpu.VMEM((256,), jnp.int32),
            pltpu.VMEM((NTMAP,), jnp.int32),
            pltpu.VMEM((NTMAP,), jnp.int32),
            pltpu.VMEM_SHARED((NSUB * 16,), jnp.int32),
        ],
    )(i1, i2)


def _xscatter_body(xh, p1h, p2h, xsh, idx_v, rows_v, sem):
    c = lax.axis_index("c")
    s = lax.axis_index("s")
    wid = s * NCORE + c
    k_is1 = wid >= 16
    t0 = jnp.where(k_is1, (wid - 16) * 128, wid * 128)

    @pl.when(k_is1)
    def _():
        pltpu.sync_copy(p2h.at[pl.ds(t0, 128)], idx_v)

    @pl.when(jnp.logical_not(k_is1))
    def _():
        pltpu.sync_copy(p1h.at[pl.ds(t0, 128)], idx_v)

    pltpu.sync_copy(xh.at[pl.ds(t0, 128)], rows_v)
    pltpu.async_copy(rows_v, xsh.at[idx_v], sem).wait()


def _xscatter(x, pos1, pos2):
    return pl.kernel(
        _xscatter_body,
        out_type=jax.ShapeDtypeStruct((SP, D), jnp.float32),
        mesh=_mesh(),
        compiler_params=pltpu.CompilerParams(needs_layout_passes=False),
        scratch_types=[
            pltpu.VMEM((128,), jnp.int32),
            pltpu.VMEM((128, D), jnp.float32),
            pltpu.SemaphoreType.DMA,
        ],
    )(x, pos1, pos2)


def _ffn_body(te_ref, ta_ref, x_ref, w1_ref, b1_ref, w2_ref, b2_ref, out_ref):
    j = pl.program_id(0)

    dn = (((1,), (0,)), ((), ()))

    @pl.when(ta_ref[j] == 1)
    def _():
        h = jax.lax.dot_general(x_ref[...], w1_ref[0], dimension_numbers=dn,
                                precision=jax.lax.Precision.DEFAULT,
                                preferred_element_type=jnp.float32)
        h = jax.nn.gelu(h + b1_ref[0])
        o = jax.lax.dot_general(h, w2_ref[0], dimension_numbers=dn,
                                precision=jax.lax.Precision.DEFAULT,
                                preferred_element_type=jnp.float32)
        out_ref[...] = o + b2_ref[0]


def _ffn_grouped(xs, te, ta, W1b, b1, W2b, b2):
    grid_spec = pltpu.PrefetchScalarGridSpec(
        num_scalar_prefetch=2,
        grid=(NT,),
        in_specs=[
            pl.BlockSpec((RB, D), lambda j, te, ta: (j, 0)),
            pl.BlockSpec((1, D, F), lambda j, te, ta: (te[j], 0, 0)),
            pl.BlockSpec((1, 1, F), lambda j, te, ta: (te[j], 0, 0)),
            pl.BlockSpec((1, F, D), lambda j, te, ta: (te[j], 0, 0)),
            pl.BlockSpec((1, 1, D), lambda j, te, ta: (te[j], 0, 0)),
        ],
        out_specs=pl.BlockSpec((RB, D), lambda j, te, ta: (j, 0)),
    )
    return pl.pallas_call(
        _ffn_body,
        grid_spec=grid_spec,
        out_shape=jax.ShapeDtypeStruct((SP, D), jnp.float32),
    )(te, ta, xs, W1b, b1.reshape(E, 1, F), W2b, b2.reshape(E, 1, D))


def _combine_body(oh, p1h, p2h, g1h, g2h, yh,
                  pa, pb, ga, gb, A, B, sem_a, sem_b):
    c = lax.axis_index("c")
    s = lax.axis_index("s")
    wid = s * NCORE + c
    t0 = wid * (T // 32)
    n = T // 32
    pltpu.sync_copy(p1h.at[pl.ds(t0, n)], pa)
    pltpu.sync_copy(p2h.at[pl.ds(t0, n)], pb)
    pltpu.sync_copy(g1h.at[pl.ds(t0, n)], ga)
    pltpu.sync_copy(g2h.at[pl.ds(t0, n)], gb)
    cpa = pltpu.async_copy(oh.at[pa], A, sem_a)
    cpb = pltpu.async_copy(oh.at[pb], B, sem_b)
    cpa.wait()
    cpb.wait()

    def body(g, carry):
        ga16 = ga[pl.ds(g * 16, 16)]
        gb16 = gb[pl.ds(g * 16, 16)]
        for tk in range(16):
            i = g * 16 + tk
            gav = jnp.full((16,), ga16[tk])
            gbv = jnp.full((16,), gb16[tk])
            for jj in range(D // 16):
                sl = pl.ds(jj * 16, 16)
                A[i, sl] = gav * A[i, sl] + gbv * B[i, sl]
        return carry

    lax.fori_loop(0, n // 16, body, 0)
    pltpu.sync_copy(A, yh.at[pl.ds(t0, n)])


def _combine(outs, pos1, pos2, g1, g2):
    n = T // 32
    return pl.kernel(
        _combine_body,
        out_type=jax.ShapeDtypeStruct((T, D), jnp.float32),
        mesh=_mesh(),
        compiler_params=pltpu.CompilerParams(needs_layout_passes=False),
        scratch_types=[
            pltpu.VMEM((n,), jnp.int32),
            pltpu.VMEM((n,), jnp.int32),
            pltpu.VMEM((n,), jnp.float32),
            pltpu.VMEM((n,), jnp.float32),
            pltpu.VMEM((n, D), jnp.float32),
            pltpu.VMEM((n, D), jnp.float32),
            pltpu.SemaphoreType.DMA,
            pltpu.SemaphoreType.DMA,
        ],
    )(outs, pos1, pos2, g1, g2)


def kernel(x, ln_scale, ln_bias, Wr, br, W1, b1, W2, b2):
    i1, i2, g1, g2 = _routing(x, ln_scale, ln_bias, Wr, br)
    i1 = i1.reshape(T)
    i2 = i2.reshape(T)
    pos1, pos2, te, ta = _dispatch(i1, i2)
    xs = _xscatter(x, pos1, pos2)
    outs = _ffn_grouped(xs, te, ta, W1, b1, W2, b2)
    return _combine(outs, pos1, pos2, g1.reshape(T), g2.reshape(T))


# speedup vs baseline: 1.6574x; 1.0141x over previous
import functools

import jax
import jax.numpy as jnp
from jax import lax
from jax.experimental import pallas as pl
from jax.experimental.pallas import tpu as pltpu
from jax.experimental.pallas import tpu_sc as plsc

T = 2048
D = 768
F = 3072
E = 8
RB = 256
NT = 24
SP = NT * RB
NTMAP = 48
NSUB = 16
NCORE = 2

_mesh = functools.partial(plsc.VectorSubcoreMesh, core_axis_name="c",
                          subcore_axis_name="s", num_cores=NCORE,
                          num_subcores=NSUB)


def _routing_body(x_ref, ls_ref, lb_ref, wr_ref, br_ref,
                  i1_ref, i2_ref, g1_ref, g2_ref):
    x = x_ref[...]
    mu = jnp.mean(x, axis=1, keepdims=True)
    xc = x - mu
    var = jnp.mean(xc * xc, axis=1, keepdims=True)
    h = xc / jnp.sqrt(var + 1e-5) * ls_ref[...] + lb_ref[...]
    logits = jnp.dot(h.astype(jnp.bfloat16), wr_ref[...].astype(jnp.bfloat16),
                     preferred_element_type=jnp.float32) + br_ref[...]
    iota_e = jax.lax.broadcasted_iota(jnp.int32, logits.shape, 1)
    v1 = jnp.max(logits, axis=1, keepdims=True)
    i1 = jnp.argmax(logits, axis=1)[:, None]
    masked = jnp.where(iota_e == i1, -jnp.inf, logits)
    v2 = jnp.max(masked, axis=1, keepdims=True)
    i2 = jnp.argmax(masked, axis=1)[:, None]
    t = jnp.exp(v2 - v1)
    i1_ref[...] = i1.reshape(-1)
    i2_ref[...] = i2.reshape(-1)
    g1_ref[...] = (1.0 / (1.0 + t)).reshape(-1)
    g2_ref[...] = (t / (1.0 + t)).reshape(-1)


def _routing(x, ln_scale, ln_bias, Wr, br):
    tb = T // 8
    return pl.pallas_call(
        _routing_body,
        grid=(8,),
        in_specs=[
            pl.BlockSpec((tb, D), lambda i: (i, 0)),
            pl.BlockSpec((1, D), lambda i: (0, 0)),
            pl.BlockSpec((1, D), lambda i: (0, 0)),
            pl.BlockSpec((D, E), lambda i: (0, 0)),
            pl.BlockSpec((1, E), lambda i: (0, 0)),
        ],
        out_specs=[
            pl.BlockSpec((tb,), lambda i: (i,)),
            pl.BlockSpec((tb,), lambda i: (i,)),
            pl.BlockSpec((tb,), lambda i: (i,)),
            pl.BlockSpec((tb,), lambda i: (i,)),
        ],
        out_shape=[
            jax.ShapeDtypeStruct((T,), jnp.int32),
            jax.ShapeDtypeStruct((T,), jnp.int32),
            jax.ShapeDtypeStruct((T,), jnp.float32),
            jax.ShapeDtypeStruct((T,), jnp.float32),
        ],
    )(x, ln_scale.reshape(1, D), ln_bias.reshape(1, D), Wr, br.reshape(1, E))


def _dispatch_body(i1h, i2h, p1h, p2h, teh, tah,
                   ex_v, cnt_v, call_v, pos_v, tile_v, act_v, sh_cnt):
    c = lax.axis_index("c")
    s = lax.axis_index("s")

    @pl.when(c == 0)
    def _():
        k_is1 = s >= 8
        t0 = jnp.where(k_is1, (s - 8) * 256, s * 256)

        @pl.when(k_is1)
        def _():
            pltpu.sync_copy(i2h.at[pl.ds(t0, 256)], ex_v)

        @pl.when(jnp.logical_not(k_is1))
        def _():
            pltpu.sync_copy(i1h.at[pl.ds(t0, 256)], ex_v)

        lane = lax.iota(jnp.int32, 16)
        zeros = jnp.zeros((16,), jnp.int32)
        counts = zeros
        vregs = []
        for i in range(16):
            v = ex_v[pl.ds(i * 16, 16)]
            vregs.append(v)
            for e in range(E):
                m = v == e
                cpop = jnp.sum(jnp.where(m, 1, 0))
                counts = counts + jnp.where(lane == e, cpop, 0)
        cnt_v[...] = counts
        pltpu.sync_copy(cnt_v, sh_cnt.at[pl.ds(s * 16, 16)])
        plsc.subcore_barrier()
        pltpu.sync_copy(sh_cnt, call_v)

        total = zeros
        prefix = zeros
        for w in range(NSUB):
            row = call_v[pl.ds(w * 16, 16)]
            total = total + row
            wmask = jnp.full((16,), w, jnp.int32) < s
            prefix = prefix + jnp.where(wmask, row, 0)
        ntiles = (total + (RB - 1)) // RB
        cum = plsc.cumsum(ntiles)
        base = (cum - ntiles) * RB
        running = base + prefix

        for i in range(16):
            v = vregs[i]
            posv = zeros
            for e in range(E):
                m = v == e
                rank = plsc.cumsum(jnp.where(m, 1, 0))
                b_e = jnp.sum(jnp.where(lane == e, running, 0))
                posv = jnp.where(m, b_e + rank - 1, posv)
                cpop = jnp.sum(jnp.where(m, 1, 0))
                running = running + jnp.where(lane == e, cpop, 0)
            pos_v[pl.ds(i * 16, 16)] = posv

        @pl.when(k_is1)
        def _():
            pltpu.sync_copy(pos_v, p2h.at[pl.ds(t0, 256)])

        @pl.when(jnp.logical_not(k_is1))
        def _():
            pltpu.sync_copy(pos_v, p1h.at[pl.ds(t0, 256)])

        @pl.when(s == 0)
        def _():
            tot_tiles = jnp.sum(jnp.where(lane == 7, cum, 0))
            cs = [jnp.sum(jnp.where(lane == e, cum, 0)) for e in range(E)]
            el = jnp.int32(0)
            for e in range(E):
                el = el + jnp.where(tot_tiles - 1 >= cs[e], 1, 0)
            el = jnp.minimum(el, E - 1)
            for ch in range(NTMAP // 16):
                jv = lane + ch * 16
                acc = zeros
                for e in range(E):
                    acc = acc + jnp.where(jv >= cs[e], 1, 0)
                active = jnp.where(jv < tot_tiles, 1, 0)
                expert = jnp.where(jv < tot_tiles, jnp.minimum(acc, E - 1), el)
                tile_v[pl.ds(ch * 16, 16)] = expert
                act_v[pl.ds(ch * 16, 16)] = active
            pltpu.sync_copy(tile_v, teh)
            pltpu.sync_copy(act_v, tah)


def _dispatch(i1, i2):
    return pl.kernel(
        _dispatch_body,
        out_type=[
            jax.ShapeDtypeStruct((T,), jnp.int32),
            jax.ShapeDtypeStruct((T,), jnp.int32),
            jax.ShapeDtypeStruct((NTMAP,), jnp.int32),
            jax.ShapeDtypeStruct((NTMAP,), jnp.int32),
        ],
        mesh=_mesh(),
        compiler_params=pltpu.CompilerParams(needs_layout_passes=False),
        scratch_types=[
            pltpu.VMEM((256,), jnp.int32),
            pltpu.VMEM((16,), jnp.int32),
            pltpu.VMEM((NSUB * 16,), jnp.int32),
            pltpu.VMEM((256,), jnp.int32),
            pltpu.VMEM((NTMAP,), jnp.int32),
            pltpu.VMEM((NTMAP,), jnp.int32),
            pltpu.VMEM_SHARED((NSUB * 16,), jnp.int32),
        ],
    )(i1, i2)


def _xscatter_body(xh, p1h, p2h, xsh, idx_v, rows_v, sem):
    c = lax.axis_index("c")
    s = lax.axis_index("s")
    wid = s * NCORE + c
    k_is1 = wid >= 16
    t0 = jnp.where(k_is1, (wid - 16) * 128, wid * 128)

    @pl.when(k_is1)
    def _():
        pltpu.sync_copy(p2h.at[pl.ds(t0, 128)], idx_v)

    @pl.when(jnp.logical_not(k_is1))
    def _():
        pltpu.sync_copy(p1h.at[pl.ds(t0, 128)], idx_v)

    pltpu.sync_copy(xh.at[pl.ds(t0, 128)], rows_v)
    pltpu.async_copy(rows_v, xsh.at[idx_v], sem).wait()


def _xscatter(x, pos1, pos2):
    return pl.kernel(
        _xscatter_body,
        out_type=jax.ShapeDtypeStruct((SP, D), jnp.float32),
        mesh=_mesh(),
        compiler_params=pltpu.CompilerParams(needs_layout_passes=False),
        scratch_types=[
            pltpu.VMEM((128,), jnp.int32),
            pltpu.VMEM((128, D), jnp.float32),
            pltpu.SemaphoreType.DMA,
        ],
    )(x, pos1, pos2)


def _ffn_body(te_ref, ta_ref, x_ref, w1_ref, b1_ref, w2_ref, b2_ref, out_ref):
    j = pl.program_id(0)

    dn = (((1,), (0,)), ((), ()))

    @pl.when(ta_ref[j] == 1)
    def _():
        h = jax.lax.dot_general(x_ref[...], w1_ref[0], dimension_numbers=dn,
                                precision=jax.lax.Precision.DEFAULT,
                                preferred_element_type=jnp.float32)
        h = jax.nn.gelu(h + b1_ref[0])
        o = jax.lax.dot_general(h, w2_ref[0], dimension_numbers=dn,
                                precision=jax.lax.Precision.DEFAULT,
                                preferred_element_type=jnp.float32)
        out_ref[...] = o + b2_ref[0]


def _ffn_grouped(xs, te, ta, W1b, b1, W2b, b2):
    grid_spec = pltpu.PrefetchScalarGridSpec(
        num_scalar_prefetch=2,
        grid=(NT,),
        in_specs=[
            pl.BlockSpec((RB, D), lambda j, te, ta: (j, 0)),
            pl.BlockSpec((1, D, F), lambda j, te, ta: (te[j], 0, 0)),
            pl.BlockSpec((1, 1, F), lambda j, te, ta: (te[j], 0, 0)),
            pl.BlockSpec((1, F, D), lambda j, te, ta: (te[j], 0, 0)),
            pl.BlockSpec((1, 1, D), lambda j, te, ta: (te[j], 0, 0)),
        ],
        out_specs=pl.BlockSpec((RB, D), lambda j, te, ta: (j, 0)),
    )
    return pl.pallas_call(
        _ffn_body,
        grid_spec=grid_spec,
        out_shape=jax.ShapeDtypeStruct((SP, D), jnp.float32),
    )(te, ta, xs, W1b, b1.reshape(E, 1, F), W2b, b2.reshape(E, 1, D))


def _combine_body(oh, p1h, p2h, g1h, g2h, yh,
                  pa, pb, ga, gb, A, B, sem_a, sem_b):
    c = lax.axis_index("c")
    s = lax.axis_index("s")
    wid = s * NCORE + c
    t0 = wid * (T // 32)
    n = T // 32
    pltpu.sync_copy(p1h.at[pl.ds(t0, n)], pa)
    pltpu.sync_copy(p2h.at[pl.ds(t0, n)], pb)
    pltpu.sync_copy(g1h.at[pl.ds(t0, n)], ga)
    pltpu.sync_copy(g2h.at[pl.ds(t0, n)], gb)
    cpa = pltpu.async_copy(oh.at[pa], A, sem_a)
    cpb = pltpu.async_copy(oh.at[pb], B, sem_b)
    cpa.wait()
    cpb.wait()

    def body(g, carry):
        ga16 = ga[pl.ds(g * 16, 16)]
        gb16 = gb[pl.ds(g * 16, 16)]
        for tk in range(16):
            i = g * 16 + tk
            gav = jnp.full((16,), ga16[tk])
            gbv = jnp.full((16,), gb16[tk])
            for jj in range(D // 16):
                sl = pl.ds(jj * 16, 16)
                A[i, sl] = gav * A[i, sl] + gbv * B[i, sl]
        return carry

    lax.fori_loop(0, n // 16, body, 0)
    pltpu.sync_copy(A, yh.at[pl.ds(t0, n)])


def _combine(outs, pos1, pos2, g1, g2):
    n = T // 32
    return pl.kernel(
        _combine_body,
        out_type=jax.ShapeDtypeStruct((T, D), jnp.float32),
        mesh=_mesh(),
        compiler_params=pltpu.CompilerParams(needs_layout_passes=False),
        scratch_types=[
            pltpu.VMEM((n,), jnp.int32),
            pltpu.VMEM((n,), jnp.int32),
            pltpu.VMEM((n,), jnp.float32),
            pltpu.VMEM((n,), jnp.float32),
            pltpu.VMEM((n, D), jnp.float32),
            pltpu.VMEM((n, D), jnp.float32),
            pltpu.SemaphoreType.DMA,
            pltpu.SemaphoreType.DMA,
        ],
    )(outs, pos1, pos2, g1, g2)


def kernel(x, ln_scale, ln_bias, Wr, br, W1, b1, W2, b2):
    i1, i2, g1, g2 = _routing(x, ln_scale, ln_bias, Wr, br)
    pos1, pos2, te, ta = _dispatch(i1, i2)
    xs = _xscatter(x, pos1, pos2)
    outs = _ffn_grouped(xs, te, ta, W1, b1, W2, b2)
    return _combine(outs, pos1, pos2, g1, g2)


# speedup vs baseline: 1.6915x; 1.0206x over previous
import functools

import jax
import jax.numpy as jnp
from jax import lax
from jax.experimental import pallas as pl
from jax.experimental.pallas import tpu as pltpu
from jax.experimental.pallas import tpu_sc as plsc

T = 2048
D = 768
F = 3072
E = 8
RB = 256
NT = 24
SP = NT * RB
NTMAP = 48
NSUB = 16
NCORE = 2

_mesh = functools.partial(plsc.VectorSubcoreMesh, core_axis_name="c",
                          subcore_axis_name="s", num_cores=NCORE,
                          num_subcores=NSUB)


def _routing_body(x_ref, ls_ref, lb_ref, wr_ref, br_ref,
                  i1_ref, i2_ref, g1_ref, g2_ref):
    x = x_ref[...]
    mu = jnp.mean(x, axis=1, keepdims=True)
    xc = x - mu
    var = jnp.mean(xc * xc, axis=1, keepdims=True)
    h = xc / jnp.sqrt(var + 1e-5) * ls_ref[...] + lb_ref[...]
    logits = jnp.dot(h.astype(jnp.bfloat16), wr_ref[...].astype(jnp.bfloat16),
                     preferred_element_type=jnp.float32) + br_ref[...]
    iota_e = jax.lax.broadcasted_iota(jnp.int32, logits.shape, 1)
    v1 = jnp.max(logits, axis=1, keepdims=True)
    i1 = jnp.argmax(logits, axis=1)[:, None]
    masked = jnp.where(iota_e == i1, -jnp.inf, logits)
    v2 = jnp.max(masked, axis=1, keepdims=True)
    i2 = jnp.argmax(masked, axis=1)[:, None]
    t = jnp.exp(v2 - v1)
    i1_ref[...] = i1.reshape(-1)
    i2_ref[...] = i2.reshape(-1)
    g1_ref[...] = (1.0 / (1.0 + t)).reshape(-1)
    g2_ref[...] = (t / (1.0 + t)).reshape(-1)


def _routing(x, ln_scale, ln_bias, Wr, br):
    tb = T // 8
    return pl.pallas_call(
        _routing_body,
        grid=(8,),
        in_specs=[
            pl.BlockSpec((tb, D), lambda i: (i, 0)),
            pl.BlockSpec((1, D), lambda i: (0, 0)),
            pl.BlockSpec((1, D), lambda i: (0, 0)),
            pl.BlockSpec((D, E), lambda i: (0, 0)),
            pl.BlockSpec((1, E), lambda i: (0, 0)),
        ],
        out_specs=[
            pl.BlockSpec((tb,), lambda i: (i,)),
            pl.BlockSpec((tb,), lambda i: (i,)),
            pl.BlockSpec((tb,), lambda i: (i,)),
            pl.BlockSpec((tb,), lambda i: (i,)),
        ],
        out_shape=[
            jax.ShapeDtypeStruct((T,), jnp.int32),
            jax.ShapeDtypeStruct((T,), jnp.int32),
            jax.ShapeDtypeStruct((T,), jnp.float32),
            jax.ShapeDtypeStruct((T,), jnp.float32),
        ],
    )(x, ln_scale.reshape(1, D), ln_bias.reshape(1, D), Wr, br.reshape(1, E))


def _dispatch_body(xh, i1h, i2h, p1h, p2h, teh, tah, xsh,
                   ex_v, cnt_v, call_v, pos_v, tile_v, act_v, idx_v, rows_v,
                   sem, sh_cnt):
    c = lax.axis_index("c")
    s = lax.axis_index("s")

    if True:
        k_is1 = s >= 8
        t0 = jnp.where(k_is1, (s - 8) * 256, s * 256)

        @pl.when(k_is1)
        def _():
            pltpu.sync_copy(i2h.at[pl.ds(t0, 256)], ex_v)

        @pl.when(jnp.logical_not(k_is1))
        def _():
            pltpu.sync_copy(i1h.at[pl.ds(t0, 256)], ex_v)

        lane = lax.iota(jnp.int32, 16)
        zeros = jnp.zeros((16,), jnp.int32)
        counts = zeros
        vregs = []
        for i in range(16):
            v = ex_v[pl.ds(i * 16, 16)]
            vregs.append(v)
            for e in range(E):
                m = v == e
                cpop = jnp.sum(jnp.where(m, 1, 0))
                counts = counts + jnp.where(lane == e, cpop, 0)
        cnt_v[...] = counts
        pltpu.sync_copy(cnt_v, sh_cnt.at[pl.ds(s * 16, 16)])
        plsc.subcore_barrier()
        pltpu.sync_copy(sh_cnt, call_v)

        total = zeros
        prefix = zeros
        for w in range(NSUB):
            row = call_v[pl.ds(w * 16, 16)]
            total = total + row
            wmask = jnp.full((16,), w, jnp.int32) < s
            prefix = prefix + jnp.where(wmask, row, 0)
        ntiles = (total + (RB - 1)) // RB
        cum = plsc.cumsum(ntiles)
        base = (cum - ntiles) * RB
        running = base + prefix

        for i in range(16):
            v = vregs[i]
            posv = zeros
            for e in range(E):
                m = v == e
                rank = plsc.cumsum(jnp.where(m, 1, 0))
                b_e = jnp.sum(jnp.where(lane == e, running, 0))
                posv = jnp.where(m, b_e + rank - 1, posv)
                cpop = jnp.sum(jnp.where(m, 1, 0))
                running = running + jnp.where(lane == e, cpop, 0)
            pos_v[pl.ds(i * 16, 16)] = posv

        @pl.when((c == 0) & k_is1)
        def _():
            pltpu.sync_copy(pos_v, p2h.at[pl.ds(t0, 256)])

        @pl.when((c == 0) & jnp.logical_not(k_is1))
        def _():
            pltpu.sync_copy(pos_v, p1h.at[pl.ds(t0, 256)])

        for i in range(8):
            idx_v[pl.ds(i * 16, 16)] = pos_v[pl.ds(c * 128 + i * 16, 16)]
        pltpu.sync_copy(xh.at[pl.ds(t0 + c * 128, 128)], rows_v)
        pltpu.async_copy(rows_v, xsh.at[idx_v], sem).wait()

        @pl.when((c == 0) & (s == 0))
        def _():
            tot_tiles = jnp.sum(jnp.where(lane == 7, cum, 0))
            cs = [jnp.sum(jnp.where(lane == e, cum, 0)) for e in range(E)]
            el = jnp.int32(0)
            for e in range(E):
                el = el + jnp.where(tot_tiles - 1 >= cs[e], 1, 0)
            el = jnp.minimum(el, E - 1)
            for ch in range(NTMAP // 16):
                jv = lane + ch * 16
                acc = zeros
                for e in range(E):
                    acc = acc + jnp.where(jv >= cs[e], 1, 0)
                active = jnp.where(jv < tot_tiles, 1, 0)
                expert = jnp.where(jv < tot_tiles, jnp.minimum(acc, E - 1), el)
                tile_v[pl.ds(ch * 16, 16)] = expert
                act_v[pl.ds(ch * 16, 16)] = active
            pltpu.sync_copy(tile_v, teh)
            pltpu.sync_copy(act_v, tah)


def _dispatch(x, i1, i2):
    return pl.kernel(
        _dispatch_body,
        out_type=[
            jax.ShapeDtypeStruct((T,), jnp.int32),
            jax.ShapeDtypeStruct((T,), jnp.int32),
            jax.ShapeDtypeStruct((NTMAP,), jnp.int32),
            jax.ShapeDtypeStruct((NTMAP,), jnp.int32),
            jax.ShapeDtypeStruct((SP, D), jnp.float32),
        ],
        mesh=_mesh(),
        compiler_params=pltpu.CompilerParams(needs_layout_passes=False),
        scratch_types=[
            pltpu.VMEM((256,), jnp.int32),
            pltpu.VMEM((16,), jnp.int32),
            pltpu.VMEM((NSUB * 16,), jnp.int32),
            pltpu.VMEM((256,), jnp.int32),
            pltpu.VMEM((NTMAP,), jnp.int32),
            pltpu.VMEM((NTMAP,), jnp.int32),
            pltpu.VMEM((128,), jnp.int32),
            pltpu.VMEM((128, D), jnp.float32),
            pltpu.SemaphoreType.DMA,
            pltpu.VMEM_SHARED((NSUB * 16,), jnp.int32),
        ],
    )(x, i1, i2)


def _ffn_body(te_ref, ta_ref, x_ref, w1_ref, b1_ref, w2_ref, b2_ref, out_ref):
    j = pl.program_id(0)

    dn = (((1,), (0,)), ((), ()))

    @pl.when(ta_ref[j] == 1)
    def _():
        h = jax.lax.dot_general(x_ref[...], w1_ref[0], dimension_numbers=dn,
                                precision=jax.lax.Precision.DEFAULT,
                                preferred_element_type=jnp.float32)
        h = jax.nn.gelu(h + b1_ref[0])
        o = jax.lax.dot_general(h, w2_ref[0], dimension_numbers=dn,
                                precision=jax.lax.Precision.DEFAULT,
                                preferred_element_type=jnp.float32)
        out_ref[...] = o + b2_ref[0]


def _ffn_grouped(xs, te, ta, W1b, b1, W2b, b2):
    grid_spec = pltpu.PrefetchScalarGridSpec(
        num_scalar_prefetch=2,
        grid=(NT,),
        in_specs=[
            pl.BlockSpec((RB, D), lambda j, te, ta: (j, 0)),
            pl.BlockSpec((1, D, F), lambda j, te, ta: (te[j], 0, 0)),
            pl.BlockSpec((1, 1, F), lambda j, te, ta: (te[j], 0, 0)),
            pl.BlockSpec((1, F, D), lambda j, te, ta: (te[j], 0, 0)),
            pl.BlockSpec((1, 1, D), lambda j, te, ta: (te[j], 0, 0)),
        ],
        out_specs=pl.BlockSpec((RB, D), lambda j, te, ta: (j, 0)),
    )
    return pl.pallas_call(
        _ffn_body,
        grid_spec=grid_spec,
        out_shape=jax.ShapeDtypeStruct((SP, D), jnp.float32),
    )(te, ta, xs, W1b, b1.reshape(E, 1, F), W2b, b2.reshape(E, 1, D))


def _combine_body(oh, p1h, p2h, g1h, g2h, yh,
                  pa, pb, ga, gb, A, B, sem_a, sem_b):
    c = lax.axis_index("c")
    s = lax.axis_index("s")
    wid = s * NCORE + c
    t0 = wid * (T // 32)
    n = T // 32
    pltpu.sync_copy(p1h.at[pl.ds(t0, n)], pa)
    pltpu.sync_copy(p2h.at[pl.ds(t0, n)], pb)
    pltpu.sync_copy(g1h.at[pl.ds(t0, n)], ga)
    pltpu.sync_copy(g2h.at[pl.ds(t0, n)], gb)
    cpa = pltpu.async_copy(oh.at[pa], A, sem_a)
    cpb = pltpu.async_copy(oh.at[pb], B, sem_b)
    cpa.wait()
    cpb.wait()

    def body(g, carry):
        ga16 = ga[pl.ds(g * 16, 16)]
        gb16 = gb[pl.ds(g * 16, 16)]
        for tk in range(16):
            i = g * 16 + tk
            gav = jnp.full((16,), ga16[tk])
            gbv = jnp.full((16,), gb16[tk])
            for jj in range(D // 16):
                sl = pl.ds(jj * 16, 16)
                A[i, sl] = gav * A[i, sl] + gbv * B[i, sl]
        return carry

    lax.fori_loop(0, n // 16, body, 0)
    pltpu.sync_copy(A, yh.at[pl.ds(t0, n)])


def _combine(outs, pos1, pos2, g1, g2):
    n = T // 32
    return pl.kernel(
        _combine_body,
        out_type=jax.ShapeDtypeStruct((T, D), jnp.float32),
        mesh=_mesh(),
        compiler_params=pltpu.CompilerParams(needs_layout_passes=False),
        scratch_types=[
            pltpu.VMEM((n,), jnp.int32),
            pltpu.VMEM((n,), jnp.int32),
            pltpu.VMEM((n,), jnp.float32),
            pltpu.VMEM((n,), jnp.float32),
            pltpu.VMEM((n, D), jnp.float32),
            pltpu.VMEM((n, D), jnp.float32),
            pltpu.SemaphoreType.DMA,
            pltpu.SemaphoreType.DMA,
        ],
    )(outs, pos1, pos2, g1, g2)


def kernel(x, ln_scale, ln_bias, Wr, br, W1, b1, W2, b2):
    i1, i2, g1, g2 = _routing(x, ln_scale, ln_bias, Wr, br)
    pos1, pos2, te, ta, xs = _dispatch(x, i1, i2)
    outs = _ffn_grouped(xs, te, ta, W1, b1, W2, b2)
    return _combine(outs, pos1, pos2, g1, g2)


# speedup vs baseline: 1.6922x; 1.0004x over previous
import functools

import jax
import jax.numpy as jnp
from jax import lax
from jax.experimental import pallas as pl
from jax.experimental.pallas import tpu as pltpu
from jax.experimental.pallas import tpu_sc as plsc

T = 2048
D = 768
F = 3072
E = 8
RB = 256
NT = 24
SP = NT * RB
NTMAP = 48
NSUB = 16
NCORE = 2

_mesh = functools.partial(plsc.VectorSubcoreMesh, core_axis_name="c",
                          subcore_axis_name="s", num_cores=NCORE,
                          num_subcores=NSUB)


def _routing_body(x_ref, ls_ref, lb_ref, wr_ref, br_ref,
                  i1_ref, i2_ref, g1_ref, g2_ref):
    x = x_ref[...]
    mu = jnp.mean(x, axis=1, keepdims=True)
    xc = x - mu
    var = jnp.mean(xc * xc, axis=1, keepdims=True)
    h = xc / jnp.sqrt(var + 1e-5) * ls_ref[...] + lb_ref[...]
    logits = jnp.dot(h.astype(jnp.bfloat16), wr_ref[...].astype(jnp.bfloat16),
                     preferred_element_type=jnp.float32) + br_ref[...]
    iota_e = jax.lax.broadcasted_iota(jnp.int32, logits.shape, 1)
    v1 = jnp.max(logits, axis=1, keepdims=True)
    i1 = jnp.argmax(logits, axis=1)[:, None]
    masked = jnp.where(iota_e == i1, -jnp.inf, logits)
    v2 = jnp.max(masked, axis=1, keepdims=True)
    i2 = jnp.argmax(masked, axis=1)[:, None]
    t = jnp.exp(v2 - v1)
    i1_ref[...] = i1.reshape(-1)
    i2_ref[...] = i2.reshape(-1)
    g1_ref[...] = (1.0 / (1.0 + t)).reshape(-1)
    g2_ref[...] = (t / (1.0 + t)).reshape(-1)


def _routing(x, ln_scale, ln_bias, Wr, br):
    tb = T // 8
    return pl.pallas_call(
        _routing_body,
        grid=(8,),
        in_specs=[
            pl.BlockSpec((tb, D), lambda i: (i, 0)),
            pl.BlockSpec((1, D), lambda i: (0, 0)),
            pl.BlockSpec((1, D), lambda i: (0, 0)),
            pl.BlockSpec((D, E), lambda i: (0, 0)),
            pl.BlockSpec((1, E), lambda i: (0, 0)),
        ],
        out_specs=[
            pl.BlockSpec((tb,), lambda i: (i,)),
            pl.BlockSpec((tb,), lambda i: (i,)),
            pl.BlockSpec((tb,), lambda i: (i,)),
            pl.BlockSpec((tb,), lambda i: (i,)),
        ],
        out_shape=[
            jax.ShapeDtypeStruct((T,), jnp.int32),
            jax.ShapeDtypeStruct((T,), jnp.int32),
            jax.ShapeDtypeStruct((T,), jnp.float32),
            jax.ShapeDtypeStruct((T,), jnp.float32),
        ],
    )(x, ln_scale.reshape(1, D), ln_bias.reshape(1, D), Wr, br.reshape(1, E))


def _dispatch_body(xh, i1h, i2h, p1h, p2h, teh, tah, xsh,
                   ex_v, cnt_v, call_v, pos_v, tile_v, act_v, idx_v, rows_v,
                   sem, sh_cnt):
    c = lax.axis_index("c")
    s = lax.axis_index("s")

    if True:
        k_is1 = s >= 8
        t0 = jnp.where(k_is1, (s - 8) * 256, s * 256)

        @pl.when(k_is1)
        def _():
            pltpu.sync_copy(i2h.at[pl.ds(t0, 256)], ex_v)

        @pl.when(jnp.logical_not(k_is1))
        def _():
            pltpu.sync_copy(i1h.at[pl.ds(t0, 256)], ex_v)

        lane = lax.iota(jnp.int32, 16)
        zeros = jnp.zeros((16,), jnp.int32)
        counts = zeros
        vregs = []
        for i in range(16):
            v = ex_v[pl.ds(i * 16, 16)]
            vregs.append(v)
            for e in range(E):
                m = v == e
                cpop = jnp.sum(jnp.where(m, 1, 0))
                counts = counts + jnp.where(lane == e, cpop, 0)
        cnt_v[...] = counts
        pltpu.sync_copy(cnt_v, sh_cnt.at[pl.ds(s * 16, 16)])
        plsc.subcore_barrier()
        pltpu.sync_copy(sh_cnt, call_v)

        total = zeros
        prefix = zeros
        for w in range(NSUB):
            row = call_v[pl.ds(w * 16, 16)]
            total = total + row
            wmask = jnp.full((16,), w, jnp.int32) < s
            prefix = prefix + jnp.where(wmask, row, 0)
        ntiles = (total + (RB - 1)) // RB
        cum = plsc.cumsum(ntiles)
        base = (cum - ntiles) * RB
        running = base + prefix

        for i in range(16):
            v = vregs[i]
            posv = zeros
            for e in range(E):
                m = v == e
                rank = plsc.cumsum(jnp.where(m, 1, 0))
                b_e = jnp.sum(jnp.where(lane == e, running, 0))
                posv = jnp.where(m, b_e + rank - 1, posv)
                cpop = jnp.sum(jnp.where(m, 1, 0))
                running = running + jnp.where(lane == e, cpop, 0)
            pos_v[pl.ds(i * 16, 16)] = posv

        @pl.when((c == 0) & k_is1)
        def _():
            pltpu.sync_copy(pos_v, p2h.at[pl.ds(t0, 256)])

        @pl.when((c == 0) & jnp.logical_not(k_is1))
        def _():
            pltpu.sync_copy(pos_v, p1h.at[pl.ds(t0, 256)])

        for i in range(8):
            idx_v[pl.ds(i * 16, 16)] = pos_v[pl.ds(c * 128 + i * 16, 16)]
        pltpu.sync_copy(xh.at[pl.ds(t0 + c * 128, 128)], rows_v)
        pltpu.async_copy(rows_v, xsh.at[idx_v], sem).wait()

        @pl.when((c == 0) & (s == 0))
        def _():
            tot_tiles = jnp.sum(jnp.where(lane == 7, cum, 0))
            cs = [jnp.sum(jnp.where(lane == e, cum, 0)) for e in range(E)]
            el = jnp.int32(0)
            for e in range(E):
                el = el + jnp.where(tot_tiles - 1 >= cs[e], 1, 0)
            el = jnp.minimum(el, E - 1)
            for ch in range(NTMAP // 16):
                jv = lane + ch * 16
                acc = zeros
                for e in range(E):
                    acc = acc + jnp.where(jv >= cs[e], 1, 0)
                active = jnp.where(jv < tot_tiles, 1, 0)
                expert = jnp.where(jv < tot_tiles, jnp.minimum(acc, E - 1), el)
                tile_v[pl.ds(ch * 16, 16)] = expert
                act_v[pl.ds(ch * 16, 16)] = active
            pltpu.sync_copy(tile_v, teh)
            pltpu.sync_copy(act_v, tah)


def _dispatch(x, i1, i2):
    return pl.kernel(
        _dispatch_body,
        out_type=[
            jax.ShapeDtypeStruct((T,), jnp.int32),
            jax.ShapeDtypeStruct((T,), jnp.int32),
            jax.ShapeDtypeStruct((NTMAP,), jnp.int32),
            jax.ShapeDtypeStruct((NTMAP,), jnp.int32),
            jax.ShapeDtypeStruct((SP, D), jnp.float32),
        ],
        mesh=_mesh(),
        compiler_params=pltpu.CompilerParams(needs_layout_passes=False),
        scratch_types=[
            pltpu.VMEM((256,), jnp.int32),
            pltpu.VMEM((16,), jnp.int32),
            pltpu.VMEM((NSUB * 16,), jnp.int32),
            pltpu.VMEM((256,), jnp.int32),
            pltpu.VMEM((NTMAP,), jnp.int32),
            pltpu.VMEM((NTMAP,), jnp.int32),
            pltpu.VMEM((128,), jnp.int32),
            pltpu.VMEM((128, D), jnp.float32),
            pltpu.SemaphoreType.DMA,
            pltpu.VMEM_SHARED((NSUB * 16,), jnp.int32),
        ],
    )(x, i1, i2)


def _ffn_body(te_ref, ta_ref, x_ref, w1_ref, b1_ref, w2_ref, b2_ref, out_ref):
    j = pl.program_id(0)

    dn = (((1,), (0,)), ((), ()))

    @pl.when(ta_ref[j] == 1)
    def _():
        h = jax.lax.dot_general(x_ref[...], w1_ref[0], dimension_numbers=dn,
                                precision=jax.lax.Precision.DEFAULT,
                                preferred_element_type=jnp.float32)
        h = jax.nn.gelu(h + b1_ref[0])
        o = jax.lax.dot_general(h, w2_ref[0], dimension_numbers=dn,
                                precision=jax.lax.Precision.DEFAULT,
                                preferred_element_type=jnp.float32)
        out_ref[...] = o + b2_ref[0]


def _ffn_grouped(xs, te, ta, W1b, b1, W2b, b2):
    grid_spec = pltpu.PrefetchScalarGridSpec(
        num_scalar_prefetch=2,
        grid=(NT,),
        in_specs=[
            pl.BlockSpec((RB, D), lambda j, te, ta: (j, 0)),
            pl.BlockSpec((1, D, F), lambda j, te, ta: (te[j], 0, 0)),
            pl.BlockSpec((1, 1, F), lambda j, te, ta: (te[j], 0, 0)),
            pl.BlockSpec((1, F, D), lambda j, te, ta: (te[j], 0, 0)),
            pl.BlockSpec((1, 1, D), lambda j, te, ta: (te[j], 0, 0)),
        ],
        out_specs=pl.BlockSpec((RB, D), lambda j, te, ta: (j, 0)),
    )
    return pl.pallas_call(
        _ffn_body,
        grid_spec=grid_spec,
        out_shape=jax.ShapeDtypeStruct((SP, D), jnp.float32),
    )(te, ta, xs, W1b, b1.reshape(E, 1, F), W2b, b2.reshape(E, 1, D))


def _combine_body(oh, p1h, p2h, g1h, g2h, yh,
                  pa, pb, ga, gb, A, B, sem_a, sem_b):
    c = lax.axis_index("c")
    s = lax.axis_index("s")
    wid = s * NCORE + c
    t0 = wid * (T // 32)
    n = T // 32
    pltpu.sync_copy(p1h.at[pl.ds(t0, n)], pa)
    pltpu.sync_copy(p2h.at[pl.ds(t0, n)], pb)
    pltpu.sync_copy(g1h.at[pl.ds(t0, n)], ga)
    pltpu.sync_copy(g2h.at[pl.ds(t0, n)], gb)
    cpa = pltpu.async_copy(oh.at[pa], A, sem_a)
    cpb = pltpu.async_copy(oh.at[pb], B, sem_b)
    cpa.wait()
    cpb.wait()

    @plsc.parallel_loop(0, n // 16, unroll=1)
    def _(g):
        ga16 = ga[pl.ds(g * 16, 16)]
        gb16 = gb[pl.ds(g * 16, 16)]
        for tk in range(16):
            i = g * 16 + tk
            gav = jnp.full((16,), ga16[tk])
            gbv = jnp.full((16,), gb16[tk])
            for jj in range(D // 16):
                sl = pl.ds(jj * 16, 16)
                A[i, sl] = gav * A[i, sl] + gbv * B[i, sl]
    pltpu.sync_copy(A, yh.at[pl.ds(t0, n)])


def _combine(outs, pos1, pos2, g1, g2):
    n = T // 32
    return pl.kernel(
        _combine_body,
        out_type=jax.ShapeDtypeStruct((T, D), jnp.float32),
        mesh=_mesh(),
        compiler_params=pltpu.CompilerParams(needs_layout_passes=False),
        scratch_types=[
            pltpu.VMEM((n,), jnp.int32),
            pltpu.VMEM((n,), jnp.int32),
            pltpu.VMEM((n,), jnp.float32),
            pltpu.VMEM((n,), jnp.float32),
            pltpu.VMEM((n, D), jnp.float32),
            pltpu.VMEM((n, D), jnp.float32),
            pltpu.SemaphoreType.DMA,
            pltpu.SemaphoreType.DMA,
        ],
    )(outs, pos1, pos2, g1, g2)


def kernel(x, ln_scale, ln_bias, Wr, br, W1, b1, W2, b2):
    i1, i2, g1, g2 = _routing(x, ln_scale, ln_bias, Wr, br)
    pos1, pos2, te, ta, xs = _dispatch(x, i1, i2)
    outs = _ffn_grouped(xs, te, ta, W1, b1, W2, b2)
    return _combine(outs, pos1, pos2, g1, g2)
